# Initial kernel scaffold; baseline (speedup 1.0000x reference)
import jax
import jax.numpy as jnp
from jax.experimental import pallas as pl


def kernel(x, edge_index, W1, b1, W2, b2, W3, b3, Wv, bv):
    raise NotImplementedError("write your pallas kernel here")


# speedup vs baseline: 30.0731x; 30.0731x over previous
import functools

import jax
import jax.numpy as jnp
from jax import lax
from jax.experimental import pallas as pl
from jax.experimental.pallas import tpu as pltpu
from jax.experimental.pallas import tpu_sc as plsc

NC = 2
NS = 16
NW = NC * NS
G = 128


def _mesh():
    return plsc.VectorSubcoreMesh(
        core_axis_name="c", subcore_axis_name="s", num_cores=NC, num_subcores=NS
    )


_SC_PARAMS = pltpu.CompilerParams(use_tc_tiling_on_sc=False)


def _make_deg(NP, GP):
    RPS = NP // NS

    @functools.partial(
        pl.kernel,
        mesh=_mesh(),
        compiler_params=_SC_PARAMS,
        out_type=jax.ShapeDtypeStruct((NC, NP), jnp.float32),
        scratch_types=[
            pltpu.VMEM((GP, G), jnp.int32),
            pltpu.VMEM((G,), jnp.float32),
            pltpu.VMEM((RPS,), jnp.float32),
            pltpu.VMEM_SHARED((NP,), jnp.float32),
        ],
    )
    def k(dst_hbm, out_hbm, dstb, ones, zbuf, acc):
        ci = lax.axis_index("c")
        si = lax.axis_index("s")
        wid = ci * NS + si

        def fill(i, _):
            zbuf[pl.ds(i * 16, 16)] = jnp.zeros((16,), jnp.float32)
            return 0

        lax.fori_loop(0, RPS // 16, fill, 0)

        def fill2(i, _):
            ones[pl.ds(i * 16, 16)] = jnp.ones((16,), jnp.float32)
            return 0

        lax.fori_loop(0, G // 16, fill2, 0)
        pltpu.sync_copy(zbuf, acc.at[pl.ds(si * RPS, RPS)])
        plsc.subcore_barrier()
        pltpu.sync_copy(dst_hbm.at[pl.ds(wid * GP, GP)], dstb)

        def body(g, _):
            pltpu.sync_copy(ones, acc.at[dstb.at[g]], add=True)
            return 0

        lax.fori_loop(0, GP, body, 0)
        plsc.subcore_barrier()
        pltpu.sync_copy(acc.at[pl.ds(si * RPS, RPS)],
                        out_hbm.at[ci, pl.ds(si * RPS, RPS)])

    return k


def _make_rowsum(N, H, NP, GP):
    RPS = NP // NS

    @functools.partial(
        pl.kernel,
        mesh=_mesh(),
        compiler_params=_SC_PARAMS,
        out_type=jax.ShapeDtypeStruct((NC, NP, H), jnp.float32),
        scratch_types=[
            pltpu.VMEM((GP, G), jnp.int32),
            pltpu.VMEM((GP, G), jnp.int32),
            pltpu.VMEM((G, H), jnp.float32),
            pltpu.VMEM((RPS, H), jnp.float32),
            pltpu.VMEM_SHARED((NP, H), jnp.float32),
            pltpu.SemaphoreType.DMA,
        ],
    )
    def k(y_hbm, src_hbm, dst_hbm, out_hbm, srcb, dstb, rows, zbuf, acc, sem):
        ci = lax.axis_index("c")
        si = lax.axis_index("s")
        wid = ci * NS + si

        def zfill(i, _):
            zbuf[i, :] = jnp.zeros((H,), jnp.float32)
            return 0

        lax.fori_loop(0, RPS, zfill, 0)
        pltpu.sync_copy(zbuf, acc.at[pl.ds(si * RPS, RPS)])
        plsc.subcore_barrier()
        pltpu.sync_copy(src_hbm.at[pl.ds(wid * GP, GP)], srcb)
        pltpu.sync_copy(dst_hbm.at[pl.ds(wid * GP, GP)], dstb)

        def body(g, _):
            pltpu.async_copy(y_hbm.at[srcb.at[g]], rows, sem).wait()
            pltpu.sync_copy(rows, acc.at[dstb.at[g]], add=True)
            return 0

        lax.fori_loop(0, GP, body, 0)
        plsc.subcore_barrier()
        pltpu.sync_copy(acc.at[pl.ds(si * RPS, RPS)],
                        out_hbm.at[ci, pl.ds(si * RPS, RPS)])

    return k


def _make_scalarsum(N, NP, GP):
    RPS = NP // NS

    @functools.partial(
        pl.kernel,
        mesh=_mesh(),
        compiler_params=_SC_PARAMS,
        out_type=jax.ShapeDtypeStruct((NC, NP), jnp.float32),
        scratch_types=[
            pltpu.VMEM((GP, G), jnp.int32),
            pltpu.VMEM((GP, G), jnp.int32),
            pltpu.VMEM((G,), jnp.float32),
            pltpu.VMEM((RPS,), jnp.float32),
            pltpu.VMEM_SHARED((NP,), jnp.float32),
            pltpu.SemaphoreType.DMA,
        ],
    )
    def k(y_hbm, src_hbm, dst_hbm, out_hbm, srcb, dstb, vals, zbuf, acc, sem):
        ci = lax.axis_index("c")
        si = lax.axis_index("s")
        wid = ci * NS + si

        def zfill(i, _):
            zbuf[pl.ds(i * 16, 16)] = jnp.zeros((16,), jnp.float32)
            return 0

        lax.fori_loop(0, RPS // 16, zfill, 0)
        pltpu.sync_copy(zbuf, acc.at[pl.ds(si * RPS, RPS)])
        plsc.subcore_barrier()
        pltpu.sync_copy(src_hbm.at[pl.ds(wid * GP, GP)], srcb)
        pltpu.sync_copy(dst_hbm.at[pl.ds(wid * GP, GP)], dstb)

        def body(g, _):
            pltpu.async_copy(y_hbm.at[srcb.at[g]], vals, sem).wait()
            pltpu.sync_copy(vals, acc.at[dstb.at[g]], add=True)
            return 0

        lax.fori_loop(0, GP, body, 0)
        plsc.subcore_barrier()
        pltpu.sync_copy(acc.at[pl.ds(si * RPS, RPS)],
                        out_hbm.at[ci, pl.ds(si * RPS, RPS)])

    return k


def _stage1(degp3, x, W1, N, NP, R):
    D = x.shape[1]
    H = W1.shape[1]
    grid = N // R

    def body(deg_ref, x_ref, w_ref, y_ref, dinv_ref):
        deg = deg_ref[0] + deg_ref[1] + 1.0
        dinv = lax.rsqrt(deg)
        xw = jnp.dot(x_ref[...], w_ref[...], preferred_element_type=jnp.float32)
        y_ref[...] = xw * dinv
        dinv_ref[...] = dinv

    return pl.pallas_call(
        body,
        grid=(grid,),
        in_specs=[
            pl.BlockSpec((NC, R, 1), lambda i: (0, i, 0)),
            pl.BlockSpec((R, D), lambda i: (i, 0)),
            pl.BlockSpec((D, H), lambda i: (0, 0)),
        ],
        out_specs=[
            pl.BlockSpec((R, H), lambda i: (i, 0)),
            pl.BlockSpec((R, 1), lambda i: (i, 0)),
        ],
        out_shape=[
            jax.ShapeDtypeStruct((N, H), jnp.float32),
            jax.ShapeDtypeStruct((NP, 1), jnp.float32),
        ],
    )(degp3, x, W1)


def _stage2(S1p, y1, dinv, b1, W2, N, NP, R):
    H = W2.shape[0]
    H2 = W2.shape[1]
    grid = N // R

    def body(s_ref, y_ref, dinv_ref, b_ref, w_ref, y2_ref):
        S = s_ref[0] + s_ref[1]
        dinv = dinv_ref[...]
        h = jnp.maximum(dinv * (S + y_ref[...]) + b_ref[...], 0.0)
        y2_ref[...] = jnp.dot(h, w_ref[...], preferred_element_type=jnp.float32) * dinv

    return pl.pallas_call(
        body,
        grid=(grid,),
        in_specs=[
            pl.BlockSpec((NC, R, H), lambda i: (0, i, 0)),
            pl.BlockSpec((R, H), lambda i: (i, 0)),
            pl.BlockSpec((R, 1), lambda i: (i, 0)),
            pl.BlockSpec((1, H), lambda i: (0, 0)),
            pl.BlockSpec((H, H2), lambda i: (0, 0)),
        ],
        out_specs=pl.BlockSpec((R, H2), lambda i: (i, 0)),
        out_shape=jax.ShapeDtypeStruct((N, H2), jnp.float32),
    )(S1p, y1, dinv, b1, W2)


def _stage3(S2p, y2, dinv, b2, W3, N, NP, R):
    H = W3.shape[0]
    grid = N // R

    def body(s_ref, y_ref, dinv_ref, b_ref, w_ref, y3_ref, hs_ref):
        S = s_ref[0] + s_ref[1]
        dinv = dinv_ref[...]
        h = jnp.maximum(dinv * (S + y_ref[...]) + b_ref[...], 0.0)
        y3_ref[...] = jnp.dot(h, w_ref[...], preferred_element_type=jnp.float32) * dinv
        hs_ref[...] = jnp.sum(h, axis=0, keepdims=True)[None]

    return pl.pallas_call(
        body,
        grid=(grid,),
        in_specs=[
            pl.BlockSpec((NC, R, H), lambda i: (0, i, 0)),
            pl.BlockSpec((R, H), lambda i: (i, 0)),
            pl.BlockSpec((R, 1), lambda i: (i, 0)),
            pl.BlockSpec((1, H), lambda i: (0, 0)),
            pl.BlockSpec((H, 1), lambda i: (0, 0)),
        ],
        out_specs=[
            pl.BlockSpec((R, 1), lambda i: (i, 0)),
            pl.BlockSpec((1, 1, H), lambda i: (i, 0, 0)),
        ],
        out_shape=[
            jax.ShapeDtypeStruct((NP, 1), jnp.float32),
            jax.ShapeDtypeStruct((grid, 1, H), jnp.float32),
        ],
    )(S2p, y2, dinv, b2, W3)


def _stage4(S3r, y3r, dinvr, b3r, hs, Wv, bvr, N):
    RW, LW = y3r.shape
    KB, H = hs.shape

    def body(s_ref, y_ref, dinv_ref, b_ref, hs_ref, wv_ref, bv_ref,
             choice_ref, value_ref):
        S = s_ref[0] + s_ref[1]
        c = dinv_ref[...] * (S + y_ref[...]) + b_ref[0, 0]
        nid = (lax.broadcasted_iota(jnp.int32, (RW, LW), 0) * LW
               + lax.broadcasted_iota(jnp.int32, (RW, LW), 1))
        valid = nid < N
        c = jnp.where(valid, c, -jnp.inf)
        m = jnp.max(c)
        e = jnp.where(valid, jnp.exp(c - m), 0.0)
        choice_ref[...] = e / jnp.sum(e)
        hmean = jnp.sum(hs_ref[...], axis=0, keepdims=True) * (1.0 / N)
        value_ref[...] = (
            jnp.dot(hmean, wv_ref[...], preferred_element_type=jnp.float32)
            + bv_ref[...]
        )

    return pl.pallas_call(
        body,
        in_specs=[
            pl.BlockSpec((NC, RW, LW), lambda: (0, 0, 0)),
            pl.BlockSpec((RW, LW), lambda: (0, 0)),
            pl.BlockSpec((RW, LW), lambda: (0, 0)),
            pl.BlockSpec((1, 1), lambda: (0, 0)),
            pl.BlockSpec((KB, H), lambda: (0, 0)),
            pl.BlockSpec((H, 1), lambda: (0, 0)),
            pl.BlockSpec((1, 1), lambda: (0, 0)),
        ],
        out_specs=[
            pl.BlockSpec((RW, LW), lambda: (0, 0)),
            pl.BlockSpec((1, 1), lambda: (0, 0)),
        ],
        out_shape=[
            jax.ShapeDtypeStruct((RW, LW), jnp.float32),
            jax.ShapeDtypeStruct((1, 1), jnp.float32),
        ],
    )(S3r, y3r, dinvr, b3r, hs, Wv, bvr)


def kernel(x, edge_index, W1, b1, W2, b2, W3, b3, Wv, bv):
    N, D = x.shape
    E = edge_index.shape[1]
    H = W1.shape[1]

    NP = ((N + 1 + 2047) // 2048) * 2048
    GP = ((-(-E // (NW * G)) + 7) // 8) * 8
    EP = NW * GP * G
    R = 2000

    src = edge_index[0]
    dst = edge_index[1]
    pad = EP - E
    srcp = jnp.concatenate([src, jnp.zeros((pad,), jnp.int32)]).reshape(NW * GP, G)
    dstp = jnp.concatenate([dst, jnp.full((pad,), N, jnp.int32)]).reshape(NW * GP, G)

    degp = _make_deg(NP, GP)(dstp)
    y1, dinv = _stage1(degp[..., None], x, W1, N, NP, R)
    S1p = _make_rowsum(N, H, NP, GP)(y1, srcp, dstp)
    y2 = _stage2(S1p, y1, dinv, b1.reshape(1, H), W2, N, NP, R)
    S2p = _make_rowsum(N, H, NP, GP)(y2, srcp, dstp)
    y3, hs = _stage3(S2p, y2, dinv, b2.reshape(1, H), W3, N, NP, R)
    hs = hs.reshape(hs.shape[0], H)
    S3p = _make_scalarsum(N, NP, GP)(y3.reshape(NP), srcp, dstp)

    LW = 128
    RW = NP // LW
    choice2d, value = _stage4(
        S3p.reshape(NC, RW, LW),
        y3.reshape(RW, LW),
        dinv.reshape(RW, LW),
        b3.reshape(1, 1),
        hs,
        Wv,
        bv.reshape(1, 1),
        N,
    )
    return choice2d.reshape(NP)[:N], value.reshape(())


# speedup vs baseline: 39.7241x; 1.3209x over previous
import functools

import jax
import jax.numpy as jnp
from jax import lax
from jax.experimental import pallas as pl
from jax.experimental.pallas import tpu as pltpu
from jax.experimental.pallas import tpu_sc as plsc

NC = 2
NS = 16
NW = NC * NS
G = 128


def _mesh():
    return plsc.VectorSubcoreMesh(
        core_axis_name="c", subcore_axis_name="s", num_cores=NC, num_subcores=NS
    )


_SC_PARAMS = pltpu.CompilerParams(use_tc_tiling_on_sc=False)


def _make_deg(NP, GP):
    RPS = NP // NS

    @functools.partial(
        pl.kernel,
        mesh=_mesh(),
        compiler_params=_SC_PARAMS,
        out_type=jax.ShapeDtypeStruct((NC, NP), jnp.float32),
        scratch_types=[
            pltpu.VMEM((GP, G), jnp.int32),
            pltpu.VMEM((G,), jnp.float32),
            pltpu.VMEM((RPS,), jnp.float32),
            pltpu.VMEM_SHARED((NP,), jnp.float32),
        ],
    )
    def k(dst_hbm, out_hbm, dstb, ones, zbuf, acc):
        ci = lax.axis_index("c")
        si = lax.axis_index("s")
        wid = ci * NS + si

        def fill(i, _):
            zbuf[pl.ds(i * 16, 16)] = jnp.zeros((16,), jnp.float32)
            return 0

        lax.fori_loop(0, RPS // 16, fill, 0)

        def fill2(i, _):
            ones[pl.ds(i * 16, 16)] = jnp.ones((16,), jnp.float32)
            return 0

        lax.fori_loop(0, G // 16, fill2, 0)
        pltpu.sync_copy(zbuf, acc.at[pl.ds(si * RPS, RPS)])
        plsc.subcore_barrier()
        pltpu.sync_copy(dst_hbm.at[pl.ds(wid * GP, GP)], dstb)

        def body(g, _):
            pltpu.sync_copy(ones, acc.at[dstb.at[g]], add=True)
            return 0

        lax.fori_loop(0, GP, body, 0)
        plsc.subcore_barrier()
        pltpu.sync_copy(acc.at[pl.ds(si * RPS, RPS)],
                        out_hbm.at[ci, pl.ds(si * RPS, RPS)])

    return k


NBUF = 8


def _make_rowsum(N, H, NP, GP):
    RPS = NP // NS

    @functools.partial(
        pl.kernel,
        mesh=_mesh(),
        compiler_params=_SC_PARAMS,
        out_type=jax.ShapeDtypeStruct((NC, NP, H), jnp.float32),
        scratch_types=[
            pltpu.VMEM((GP, G), jnp.int32),
            pltpu.VMEM((GP, G), jnp.int32),
            pltpu.VMEM((NBUF, G, H), jnp.float32),
            pltpu.VMEM((RPS, H), jnp.float32),
            pltpu.VMEM_SHARED((NP, H), jnp.float32),
            pltpu.SemaphoreType.DMA((NBUF,)),
            pltpu.SemaphoreType.DMA((NBUF,)),
        ],
    )
    def k(y_hbm, src_hbm, dst_hbm, out_hbm, srcb, dstb, rows, zbuf, acc,
          gsem, ssem):
        ci = lax.axis_index("c")
        si = lax.axis_index("s")
        wid = ci * NS + si

        def zfill(i, _):
            zbuf[i, :] = jnp.zeros((H,), jnp.float32)
            return 0

        lax.fori_loop(0, RPS, zfill, 0)
        pltpu.sync_copy(src_hbm.at[pl.ds(wid * GP, GP)], srcb)
        pltpu.sync_copy(dst_hbm.at[pl.ds(wid * GP, GP)], dstb)
        for b in range(NBUF):
            pltpu.async_copy(y_hbm.at[srcb.at[b]], rows.at[b], gsem.at[b])
        pltpu.sync_copy(zbuf, acc.at[pl.ds(si * RPS, RPS)])
        plsc.subcore_barrier()

        def outer(t, _):
            t0 = t * NBUF
            for b in range(NBUF):
                g = t0 + b
                pltpu.make_async_copy(
                    y_hbm.at[srcb.at[g]], rows.at[b], gsem.at[b]).wait()
                pltpu.async_copy(rows.at[b], acc.at[dstb.at[g]], ssem.at[b],
                                 add=True)
            for b in range(NBUF):
                ng = t0 + NBUF + b

                @pl.when(ng < GP)
                def _():
                    pltpu.make_async_copy(
                        rows.at[b], acc.at[dstb.at[b]], ssem.at[b]).wait()
                    pltpu.async_copy(y_hbm.at[srcb.at[ng]], rows.at[b],
                                     gsem.at[b])

            return 0

        lax.fori_loop(0, GP // NBUF, outer, 0)
        for b in range(NBUF):
            pltpu.make_async_copy(rows.at[b], acc.at[dstb.at[b]],
                                  ssem.at[b]).wait()
        plsc.subcore_barrier()
        pltpu.sync_copy(acc.at[pl.ds(si * RPS, RPS)],
                        out_hbm.at[ci, pl.ds(si * RPS, RPS)])

    return k


def _make_scalarsum(N, NP, GP):
    RPS = NP // NS

    @functools.partial(
        pl.kernel,
        mesh=_mesh(),
        compiler_params=_SC_PARAMS,
        out_type=jax.ShapeDtypeStruct((NC, NP), jnp.float32),
        scratch_types=[
            pltpu.VMEM((GP, G), jnp.int32),
            pltpu.VMEM((GP, G), jnp.int32),
            pltpu.VMEM((NBUF, G), jnp.float32),
            pltpu.VMEM((RPS,), jnp.float32),
            pltpu.VMEM_SHARED((NP,), jnp.float32),
            pltpu.SemaphoreType.DMA((NBUF,)),
            pltpu.SemaphoreType.DMA((NBUF,)),
        ],
    )
    def k(y_hbm, src_hbm, dst_hbm, out_hbm, srcb, dstb, vals, zbuf, acc,
          gsem, ssem):
        ci = lax.axis_index("c")
        si = lax.axis_index("s")
        wid = ci * NS + si

        def zfill(i, _):
            zbuf[pl.ds(i * 16, 16)] = jnp.zeros((16,), jnp.float32)
            return 0

        lax.fori_loop(0, RPS // 16, zfill, 0)
        pltpu.sync_copy(src_hbm.at[pl.ds(wid * GP, GP)], srcb)
        pltpu.sync_copy(dst_hbm.at[pl.ds(wid * GP, GP)], dstb)
        for b in range(NBUF):
            pltpu.async_copy(y_hbm.at[srcb.at[b]], vals.at[b], gsem.at[b])
        pltpu.sync_copy(zbuf, acc.at[pl.ds(si * RPS, RPS)])
        plsc.subcore_barrier()

        def outer(t, _):
            t0 = t * NBUF
            for b in range(NBUF):
                g = t0 + b
                pltpu.make_async_copy(
                    y_hbm.at[srcb.at[g]], vals.at[b], gsem.at[b]).wait()
                pltpu.async_copy(vals.at[b], acc.at[dstb.at[g]], ssem.at[b],
                                 add=True)
            for b in range(NBUF):
                ng = t0 + NBUF + b

                @pl.when(ng < GP)
                def _():
                    pltpu.make_async_copy(
                        vals.at[b], acc.at[dstb.at[b]], ssem.at[b]).wait()
                    pltpu.async_copy(y_hbm.at[srcb.at[ng]], vals.at[b],
                                     gsem.at[b])

            return 0

        lax.fori_loop(0, GP // NBUF, outer, 0)
        for b in range(NBUF):
            pltpu.make_async_copy(vals.at[b], acc.at[dstb.at[b]],
                                  ssem.at[b]).wait()
        plsc.subcore_barrier()
        pltpu.sync_copy(acc.at[pl.ds(si * RPS, RPS)],
                        out_hbm.at[ci, pl.ds(si * RPS, RPS)])

    return k


def _stage1(degp3, x, W1, N, NP, R):
    D = x.shape[1]
    H = W1.shape[1]
    grid = N // R

    def body(deg_ref, x_ref, w_ref, y_ref, dinv_ref):
        deg = deg_ref[0] + deg_ref[1] + 1.0
        dinv = lax.rsqrt(deg)
        xw = jnp.dot(x_ref[...], w_ref[...], preferred_element_type=jnp.float32)
        y_ref[...] = xw * dinv
        dinv_ref[...] = dinv

    return pl.pallas_call(
        body,
        grid=(grid,),
        in_specs=[
            pl.BlockSpec((NC, R, 1), lambda i: (0, i, 0)),
            pl.BlockSpec((R, D), lambda i: (i, 0)),
            pl.BlockSpec((D, H), lambda i: (0, 0)),
        ],
        out_specs=[
            pl.BlockSpec((R, H), lambda i: (i, 0)),
            pl.BlockSpec((R, 1), lambda i: (i, 0)),
        ],
        out_shape=[
            jax.ShapeDtypeStruct((N, H), jnp.float32),
            jax.ShapeDtypeStruct((NP, 1), jnp.float32),
        ],
    )(degp3, x, W1)


def _stage2(S1p, y1, dinv, b1, W2, N, NP, R):
    H = W2.shape[0]
    H2 = W2.shape[1]
    grid = N // R

    def body(s_ref, y_ref, dinv_ref, b_ref, w_ref, y2_ref):
        S = s_ref[0] + s_ref[1]
        dinv = dinv_ref[...]
        h = jnp.maximum(dinv * (S + y_ref[...]) + b_ref[...], 0.0)
        y2_ref[...] = jnp.dot(h, w_ref[...], preferred_element_type=jnp.float32) * dinv

    return pl.pallas_call(
        body,
        grid=(grid,),
        in_specs=[
            pl.BlockSpec((NC, R, H), lambda i: (0, i, 0)),
            pl.BlockSpec((R, H), lambda i: (i, 0)),
            pl.BlockSpec((R, 1), lambda i: (i, 0)),
            pl.BlockSpec((1, H), lambda i: (0, 0)),
            pl.BlockSpec((H, H2), lambda i: (0, 0)),
        ],
        out_specs=pl.BlockSpec((R, H2), lambda i: (i, 0)),
        out_shape=jax.ShapeDtypeStruct((N, H2), jnp.float32),
    )(S1p, y1, dinv, b1, W2)


def _stage3(S2p, y2, dinv, b2, W3, N, NP, R):
    H = W3.shape[0]
    grid = N // R

    def body(s_ref, y_ref, dinv_ref, b_ref, w_ref, y3_ref, hs_ref):
        S = s_ref[0] + s_ref[1]
        dinv = dinv_ref[...]
        h = jnp.maximum(dinv * (S + y_ref[...]) + b_ref[...], 0.0)
        y3_ref[...] = jnp.dot(h, w_ref[...], preferred_element_type=jnp.float32) * dinv
        hs_ref[...] = jnp.sum(h, axis=0, keepdims=True)[None]

    return pl.pallas_call(
        body,
        grid=(grid,),
        in_specs=[
            pl.BlockSpec((NC, R, H), lambda i: (0, i, 0)),
            pl.BlockSpec((R, H), lambda i: (i, 0)),
            pl.BlockSpec((R, 1), lambda i: (i, 0)),
            pl.BlockSpec((1, H), lambda i: (0, 0)),
            pl.BlockSpec((H, 1), lambda i: (0, 0)),
        ],
        out_specs=[
            pl.BlockSpec((R, 1), lambda i: (i, 0)),
            pl.BlockSpec((1, 1, H), lambda i: (i, 0, 0)),
        ],
        out_shape=[
            jax.ShapeDtypeStruct((NP, 1), jnp.float32),
            jax.ShapeDtypeStruct((grid, 1, H), jnp.float32),
        ],
    )(S2p, y2, dinv, b2, W3)


def _stage4(S3r, y3r, dinvr, b3r, hs, Wv, bvr, N):
    RW, LW = y3r.shape
    KB, H = hs.shape

    def body(s_ref, y_ref, dinv_ref, b_ref, hs_ref, wv_ref, bv_ref,
             choice_ref, value_ref):
        S = s_ref[0] + s_ref[1]
        c = dinv_ref[...] * (S + y_ref[...]) + b_ref[0, 0]
        nid = (lax.broadcasted_iota(jnp.int32, (RW, LW), 0) * LW
               + lax.broadcasted_iota(jnp.int32, (RW, LW), 1))
        valid = nid < N
        c = jnp.where(valid, c, -jnp.inf)
        m = jnp.max(c)
        e = jnp.where(valid, jnp.exp(c - m), 0.0)
        choice_ref[...] = e / jnp.sum(e)
        hmean = jnp.sum(hs_ref[...], axis=0, keepdims=True) * (1.0 / N)
        value_ref[...] = (
            jnp.dot(hmean, wv_ref[...], preferred_element_type=jnp.float32)
            + bv_ref[...]
        )

    return pl.pallas_call(
        body,
        in_specs=[
            pl.BlockSpec((NC, RW, LW), lambda: (0, 0, 0)),
            pl.BlockSpec((RW, LW), lambda: (0, 0)),
            pl.BlockSpec((RW, LW), lambda: (0, 0)),
            pl.BlockSpec((1, 1), lambda: (0, 0)),
            pl.BlockSpec((KB, H), lambda: (0, 0)),
            pl.BlockSpec((H, 1), lambda: (0, 0)),
            pl.BlockSpec((1, 1), lambda: (0, 0)),
        ],
        out_specs=[
            pl.BlockSpec((RW, LW), lambda: (0, 0)),
            pl.BlockSpec((1, 1), lambda: (0, 0)),
        ],
        out_shape=[
            jax.ShapeDtypeStruct((RW, LW), jnp.float32),
            jax.ShapeDtypeStruct((1, 1), jnp.float32),
        ],
    )(S3r, y3r, dinvr, b3r, hs, Wv, bvr)


def kernel(x, edge_index, W1, b1, W2, b2, W3, b3, Wv, bv):
    N, D = x.shape
    E = edge_index.shape[1]
    H = W1.shape[1]

    NP = ((N + 1 + 2047) // 2048) * 2048
    GP = ((-(-E // (NW * G)) + 7) // 8) * 8
    EP = NW * GP * G
    R = 2000

    src = edge_index[0]
    dst = edge_index[1]
    pad = EP - E
    srcp = jnp.concatenate([src, jnp.zeros((pad,), jnp.int32)]).reshape(NW * GP, G)
    dstp = jnp.concatenate([dst, jnp.full((pad,), N, jnp.int32)]).reshape(NW * GP, G)

    degp = _make_deg(NP, GP)(dstp)
    y1, dinv = _stage1(degp[..., None], x, W1, N, NP, R)
    S1p = _make_rowsum(N, H, NP, GP)(y1, srcp, dstp)
    y2 = _stage2(S1p, y1, dinv, b1.reshape(1, H), W2, N, NP, R)
    S2p = _make_rowsum(N, H, NP, GP)(y2, srcp, dstp)
    y3, hs = _stage3(S2p, y2, dinv, b2.reshape(1, H), W3, N, NP, R)
    hs = hs.reshape(hs.shape[0], H)
    S3p = _make_scalarsum(N, NP, GP)(y3.reshape(NP), srcp, dstp)

    LW = 128
    RW = NP // LW
    choice2d, value = _stage4(
        S3p.reshape(NC, RW, LW),
        y3.reshape(RW, LW),
        dinv.reshape(RW, LW),
        b3.reshape(1, 1),
        hs,
        Wv,
        bv.reshape(1, 1),
        N,
    )
    return choice2d.reshape(NP)[:N], value.reshape(())
